# Initial kernel scaffold; baseline (speedup 1.0000x reference)
import jax
import jax.numpy as jnp
from jax.experimental import pallas as pl


def kernel(x, emb_weight):
    raise NotImplementedError("write your pallas kernel here")


# speedup vs baseline: 1.4872x; 1.4872x over previous
import jax
import jax.numpy as jnp
from jax.experimental import pallas as pl


_BS = 512


def _add_kernel(x_ref, emb_ref, o_ref):
    o_ref[0] = x_ref[0] + emb_ref[...]


def kernel(x, emb_weight):
    B, S, D = x.shape
    emb = emb_weight[:S]
    grid = (S // _BS, B)
    return pl.pallas_call(
        _add_kernel,
        grid=grid,
        in_specs=[
            pl.BlockSpec((1, _BS, D), lambda i, j: (j, i, 0)),
            pl.BlockSpec((_BS, D), lambda i, j: (i, 0)),
        ],
        out_specs=pl.BlockSpec((1, _BS, D), lambda i, j: (j, i, 0)),
        out_shape=jax.ShapeDtypeStruct((B, S, D), x.dtype),
    )(x, emb)


# speedup vs baseline: 1.6707x; 1.1234x over previous
import jax
import jax.numpy as jnp
from jax.experimental import pallas as pl


_BS = 1024


def _add_kernel(x_ref, emb_ref, o_ref):
    o_ref[0] = x_ref[0] + emb_ref[...]


def kernel(x, emb_weight):
    B, S, D = x.shape
    emb = emb_weight[:S]
    grid = (S // _BS, B)
    return pl.pallas_call(
        _add_kernel,
        grid=grid,
        in_specs=[
            pl.BlockSpec((1, _BS, D), lambda i, j: (j, i, 0)),
            pl.BlockSpec((_BS, D), lambda i, j: (i, 0)),
        ],
        out_specs=pl.BlockSpec((1, _BS, D), lambda i, j: (j, i, 0)),
        out_shape=jax.ShapeDtypeStruct((B, S, D), x.dtype),
    )(x, emb)


# speedup vs baseline: 1.7351x; 1.0385x over previous
import jax
import jax.numpy as jnp
from jax.experimental import pallas as pl


_BS = 2048


def _add_kernel(x_ref, emb_ref, o_ref):
    o_ref[0] = x_ref[0] + emb_ref[...]


def kernel(x, emb_weight):
    B, S, D = x.shape
    emb = emb_weight[:S]
    grid = (S // _BS, B)
    return pl.pallas_call(
        _add_kernel,
        grid=grid,
        in_specs=[
            pl.BlockSpec((1, _BS, D), lambda i, j: (j, i, 0)),
            pl.BlockSpec((_BS, D), lambda i, j: (i, 0)),
        ],
        out_specs=pl.BlockSpec((1, _BS, D), lambda i, j: (j, i, 0)),
        out_shape=jax.ShapeDtypeStruct((B, S, D), x.dtype),
    )(x, emb)
